# baseline (speedup 1.0000x reference)
import functools

import jax
import jax.numpy as jnp
from jax import lax
from jax.experimental import pallas as pl
from jax.experimental.pallas import tpu as pltpu
from jax.experimental.pallas import tpu_sc as plsc

N = 10000
E = 320000
G = 256
IN_F = 128
HID = 64
N_CLS = 29

NC = 2
NS = 16
NW = NC * NS
CHUNK = 40
CPT = 250
NB = 10
NP = 10240
RPT = NP // NS
BLK = 2560


@functools.cache
def _make_deg_kernel():
    mesh = plsc.VectorSubcoreMesh(core_axis_name="c", subcore_axis_name="s")

    @functools.partial(
        pl.kernel,
        mesh=mesh,
        compiler_params=pltpu.CompilerParams(use_tc_tiling_on_sc=False),
        out_type=jax.ShapeDtypeStruct((NC, NP, 16), jnp.float32),
        scratch_types=[
            pltpu.VMEM((CPT, CHUNK), jnp.int32),
            pltpu.VMEM((CHUNK, 16), jnp.float32),
            pltpu.VMEM_SHARED((NP, 16), jnp.float32),
            pltpu.SemaphoreType.DMA,
        ],
    )
    def _deg_kernel(dst_hbm, out_hbm, didx_v, ones_v, acc_sh, sem):
        kernel_body_deg(dst_hbm, out_hbm, didx_v, ones_v, acc_sh, sem)

    return _deg_kernel


def kernel_body_deg(dst_hbm, out_hbm, didx_v, ones_v, acc_sh, sem):
    cid = lax.axis_index("c")
    sid = lax.axis_index("s")
    wid = cid * NS + sid

    @pl.loop(0, CHUNK)
    def _(r):
        ones_v[r, pl.ds(0, 16)] = jnp.zeros((16,), jnp.float32)

    @pl.loop(0, RPT // CHUNK)
    def _(k):
        pltpu.sync_copy(ones_v, acc_sh.at[pl.ds(sid * RPT + k * CHUNK, CHUNK)])

    pltpu.sync_copy(dst_hbm.at[wid], didx_v)

    @pl.loop(0, CHUNK)
    def _(r):
        ones_v[r, pl.ds(0, 16)] = jnp.ones((16,), jnp.float32)

    plsc.subcore_barrier()

    for j in range(NB):
        pltpu.async_copy(ones_v, acc_sh.at[didx_v.at[j]], sem, add=True)

    @pl.loop(NB, CPT)
    def _(j):
        pltpu.make_async_copy(ones_v, acc_sh.at[didx_v.at[0]], sem).wait()
        pltpu.async_copy(ones_v, acc_sh.at[didx_v.at[j]], sem, add=True)

    for _ in range(NB):
        pltpu.make_async_copy(ones_v, acc_sh.at[didx_v.at[0]], sem).wait()

    plsc.subcore_barrier()
    pltpu.sync_copy(acc_sh.at[pl.ds(sid * RPT, RPT)],
                    out_hbm.at[cid].at[pl.ds(sid * RPT, RPT)])


@functools.cache
def _make_agg_kernel():
    mesh = plsc.VectorSubcoreMesh(core_axis_name="c", subcore_axis_name="s")

    @functools.partial(
        pl.kernel,
        mesh=mesh,
        compiler_params=pltpu.CompilerParams(use_tc_tiling_on_sc=False),
        out_type=jax.ShapeDtypeStruct((NC, NP, HID), jnp.float32),
        scratch_types=[
            pltpu.VMEM((CPT, CHUNK), jnp.int32),
            pltpu.VMEM((CPT, CHUNK), jnp.int32),
            pltpu.VMEM((NB * CHUNK, HID), jnp.float32),
        ] + [pltpu.SemaphoreType.DMA] * (2 * NB) + [
            pltpu.VMEM_SHARED((NP, HID), jnp.float32),
        ],
    )
    def _agg_kernel(g_hbm, src_hbm, dst_hbm, out_hbm, sidx_v, didx_v, rows_v,
                    *rest):
        kernel_body_agg(g_hbm, src_hbm, dst_hbm, out_hbm, sidx_v, didx_v,
                        rows_v, rest[:NB], rest[NB:2 * NB], rest[2 * NB])

    return _agg_kernel


def kernel_body_agg(g_hbm, src_hbm, dst_hbm, out_hbm, sidx_v, didx_v, rows_v,
                    gsem, ssem, acc_sh):
    cid = lax.axis_index("c")
    sid = lax.axis_index("s")
    wid = cid * NS + sid

    def buf(k):
        return rows_v.at[pl.ds(k * CHUNK, CHUNK)]

    @pl.loop(0, CHUNK)
    def _(r):
        @pl.loop(0, HID, step=16)
        def _(c0):
            rows_v[r, pl.ds(c0, 16)] = jnp.zeros((16,), jnp.float32)

    @pl.loop(0, RPT // CHUNK)
    def _(k):
        pltpu.sync_copy(buf(0), acc_sh.at[pl.ds(sid * RPT + k * CHUNK, CHUNK)])

    pltpu.sync_copy(src_hbm.at[wid], sidx_v)
    pltpu.sync_copy(dst_hbm.at[wid], didx_v)

    for k in range(NB):
        pltpu.async_copy(g_hbm.at[sidx_v.at[k]], buf(k), gsem[k])

    plsc.subcore_barrier()

    @pl.loop(0, CPT // NB)
    def _(t):
        base = t * NB
        for k in range(NB):
            pltpu.make_async_copy(g_hbm.at[sidx_v.at[0]], buf(k),
                                  gsem[k]).wait()
            pltpu.async_copy(buf(k), acc_sh.at[didx_v.at[base + k]],
                             ssem[k], add=True)
        for k in range(NB):
            @pl.when(t < CPT // NB - 1)
            def _():
                pltpu.make_async_copy(buf(k), acc_sh.at[didx_v.at[0]],
                                      ssem[k]).wait()
                pltpu.async_copy(g_hbm.at[sidx_v.at[base + NB + k]],
                                 buf(k), gsem[k])

    for k in range(NB):
        pltpu.make_async_copy(buf(k), acc_sh.at[didx_v.at[0]], ssem[k]).wait()

    plsc.subcore_barrier()
    pltpu.sync_copy(acc_sh.at[pl.ds(sid * RPT, RPT)],
                    out_hbm.at[cid].at[pl.ds(sid * RPT, RPT)])


def _row_ids(i):
    return i * BLK + lax.broadcasted_iota(jnp.int32, (BLK, 1), 0)


def _tca_body(x_ref, w1_ref, h1_ref):
    i = pl.program_id(0)
    h = lax.dot_general(x_ref[...], w1_ref[...],
                        (((1,), (1,)), ((), ())),
                        preferred_element_type=jnp.float32)
    h1_ref[...] = jnp.where(_row_ids(i) < N, h, 0.0)


def _tcb_body(deg_ref, h1_ref, g1_ref, dinv_ref):
    deg = deg_ref[0, :, :1] + deg_ref[1, :, :1] + 1.0
    dinv = lax.rsqrt(deg)
    dinv_ref[...] = jnp.broadcast_to(dinv, (BLK, 16))
    g1_ref[...] = h1_ref[...] * dinv


def _tc2_body(acc_ref, g1_ref, dinv_ref, b1_ref, w2_ref, g2_ref):
    dinv = dinv_ref[:, :1]
    z1 = jnp.maximum(dinv * (acc_ref[0] + acc_ref[1] + g1_ref[...])
                     + b1_ref[...], 0.0)
    h2 = lax.dot_general(z1, w2_ref[...], (((1,), (1,)), ((), ())),
                         preferred_element_type=jnp.float32)
    g2_ref[...] = h2 * dinv


def _tc3_body(acc_ref, g2_ref, dinv_ref, b2_ref, batch_ref,
              wfc_ref, bfc_ref, out_ref, pool_ref):
    i = pl.program_id(0)
    dinv = dinv_ref[:, :1]
    z2 = jnp.maximum(dinv * (acc_ref[0] + acc_ref[1] + g2_ref[...])
                     + b2_ref[...], 0.0)
    valid = _row_ids(i) < N
    onehot = jnp.where(
        valid,
        (batch_ref[...] == jnp.arange(G, dtype=jnp.int32)[None, :]
         ).astype(jnp.float32),
        0.0)
    ext = jnp.concatenate([z2, jnp.ones((BLK, 1), jnp.float32)], axis=1)
    part = lax.dot_general(onehot, ext, (((0,), (0,)), ((), ())),
                           preferred_element_type=jnp.float32)

    @pl.when(i == 0)
    def _():
        pool_ref[...] = part

    @pl.when(i > 0)
    def _():
        pool_ref[...] += part

    @pl.when(i == NP // BLK - 1)
    def _():
        sums = pool_ref[:, :HID]
        cnt = jnp.maximum(pool_ref[:, HID:HID + 1], 1.0)
        pooled = sums / cnt
        out_ref[...] = lax.dot_general(
            pooled, wfc_ref[...], (((1,), (1,)), ((), ())),
            preferred_element_type=jnp.float32) + bfc_ref[...]


def _row_spec(cols):
    return pl.BlockSpec((BLK, cols), lambda i: (i, 0))


def _pair_spec(cols):
    return pl.BlockSpec((NC, BLK, cols), lambda i: (0, i, 0))


def _full_spec(shape):
    return pl.BlockSpec(shape, lambda i: tuple(0 for _ in shape))


def _tca(x, w1):
    return pl.pallas_call(
        _tca_body,
        grid=(NP // BLK,),
        in_specs=[_row_spec(IN_F), _full_spec((HID, IN_F))],
        out_specs=_row_spec(HID),
        out_shape=jax.ShapeDtypeStruct((NP, HID), jnp.float32),
    )(x, w1)


def _tcb(deg, h1):
    return pl.pallas_call(
        _tcb_body,
        grid=(NP // BLK,),
        in_specs=[_pair_spec(16), _row_spec(HID)],
        out_specs=[_row_spec(HID), _row_spec(16)],
        out_shape=[jax.ShapeDtypeStruct((NP, HID), jnp.float32),
                   jax.ShapeDtypeStruct((NP, 16), jnp.float32)],
    )(deg, h1)


def _tc2(acc, g1, dinv16, b1, w2):
    return pl.pallas_call(
        _tc2_body,
        grid=(NP // BLK,),
        in_specs=[_pair_spec(HID), _row_spec(HID),
                  _row_spec(16), _full_spec((1, HID)), _full_spec((HID, HID))],
        out_specs=_row_spec(HID),
        out_shape=jax.ShapeDtypeStruct((NP, HID), jnp.float32),
    )(acc, g1, dinv16, b1, w2)


def _tc3(acc, g2, dinv16, b2, batch_col, wfc, bfc):
    return pl.pallas_call(
        _tc3_body,
        grid=(NP // BLK,),
        in_specs=[_pair_spec(HID), _row_spec(HID),
                  _row_spec(16), _full_spec((1, HID)), _row_spec(1),
                  _full_spec((N_CLS, HID)), _full_spec((1, N_CLS))],
        out_specs=_full_spec((G, N_CLS)),
        out_shape=jax.ShapeDtypeStruct((G, N_CLS), jnp.float32),
        scratch_shapes=[pltpu.VMEM((G, HID + 1), jnp.float32)],
    )(acc, g2, dinv16, b2, batch_col, wfc, bfc)


def kernel(x, edge_index, batch, W1, b1, W2, b2, Wfc, bfc):
    src_p = edge_index[0].astype(jnp.int32).reshape(NW, CPT, CHUNK)
    dst_p = edge_index[1].astype(jnp.int32).reshape(NW, CPT, CHUNK)
    batch_col = batch.astype(jnp.int32).reshape(N, 1)

    h1 = _tca(x, W1)
    deg = _make_deg_kernel()(dst_p)
    g1, dinv16 = _tcb(deg, h1)
    acc1 = _make_agg_kernel()(g1, src_p, dst_p)
    g2 = _tc2(acc1, g1, dinv16, b1.reshape(1, HID), W2)
    acc2 = _make_agg_kernel()(g2, src_p, dst_p)
    out = _tc3(acc2, g2, dinv16, b2.reshape(1, HID), batch_col,
               Wfc, bfc.reshape(1, N_CLS))
    return out


# speedup vs baseline: 1.0573x; 1.0573x over previous
import functools

import jax
import jax.numpy as jnp
from jax import lax
from jax.experimental import pallas as pl
from jax.experimental.pallas import tpu as pltpu
from jax.experimental.pallas import tpu_sc as plsc

N = 10000
E = 320000
G = 256
IN_F = 128
HID = 64
N_CLS = 29

NC = 2
NS = 16
NW = NC * NS
CHUNK = 128
CPT = 80
NB = 8
E_PAD = NW * CPT * CHUNK
NP = 10240
RPT = NP // NS
BLK = 2560


@functools.cache
def _make_deg_kernel():
    mesh = plsc.VectorSubcoreMesh(core_axis_name="c", subcore_axis_name="s")

    @functools.partial(
        pl.kernel,
        mesh=mesh,
        compiler_params=pltpu.CompilerParams(use_tc_tiling_on_sc=False),
        out_type=jax.ShapeDtypeStruct((NC, NP, 16), jnp.float32),
        scratch_types=[
            pltpu.VMEM((CPT, CHUNK), jnp.int32),
            pltpu.VMEM((CHUNK, 16), jnp.float32),
            pltpu.VMEM_SHARED((NP, 16), jnp.float32),
            pltpu.SemaphoreType.DMA,
        ],
    )
    def _deg_kernel(dst_hbm, out_hbm, didx_v, ones_v, acc_sh, sem):
        kernel_body_deg(dst_hbm, out_hbm, didx_v, ones_v, acc_sh, sem)

    return _deg_kernel


def kernel_body_deg(dst_hbm, out_hbm, didx_v, ones_v, acc_sh, sem):
    cid = lax.axis_index("c")
    sid = lax.axis_index("s")
    wid = cid * NS + sid

    @pl.loop(0, CHUNK)
    def _(r):
        ones_v[r, pl.ds(0, 16)] = jnp.zeros((16,), jnp.float32)

    @pl.loop(0, RPT // CHUNK)
    def _(k):
        pltpu.sync_copy(ones_v, acc_sh.at[pl.ds(sid * RPT + k * CHUNK, CHUNK)])

    pltpu.sync_copy(dst_hbm.at[wid], didx_v)

    @pl.loop(0, CHUNK)
    def _(r):
        ones_v[r, pl.ds(0, 16)] = jnp.ones((16,), jnp.float32)

    plsc.subcore_barrier()

    for j in range(NB):
        pltpu.async_copy(ones_v, acc_sh.at[didx_v.at[j]], sem, add=True)

    @pl.loop(NB, CPT)
    def _(j):
        pltpu.make_async_copy(ones_v, acc_sh.at[didx_v.at[0]], sem).wait()
        pltpu.async_copy(ones_v, acc_sh.at[didx_v.at[j]], sem, add=True)

    for _ in range(NB):
        pltpu.make_async_copy(ones_v, acc_sh.at[didx_v.at[0]], sem).wait()

    plsc.subcore_barrier()
    pltpu.sync_copy(acc_sh.at[pl.ds(sid * RPT, RPT)],
                    out_hbm.at[cid].at[pl.ds(sid * RPT, RPT)])


@functools.cache
def _make_agg_kernel():
    mesh = plsc.VectorSubcoreMesh(core_axis_name="c", subcore_axis_name="s")

    @functools.partial(
        pl.kernel,
        mesh=mesh,
        compiler_params=pltpu.CompilerParams(use_tc_tiling_on_sc=False),
        out_type=jax.ShapeDtypeStruct((NC, NP, HID), jnp.float32),
        scratch_types=[
            pltpu.VMEM((CPT, CHUNK), jnp.int32),
            pltpu.VMEM((CPT, CHUNK), jnp.int32),
            pltpu.VMEM((NB * CHUNK, HID), jnp.float32),
        ] + [pltpu.SemaphoreType.DMA] * (2 * NB) + [
            pltpu.VMEM_SHARED((NP, HID), jnp.float32),
        ],
    )
    def _agg_kernel(g_hbm, src_hbm, dst_hbm, out_hbm, sidx_v, didx_v, rows_v,
                    *rest):
        kernel_body_agg(g_hbm, src_hbm, dst_hbm, out_hbm, sidx_v, didx_v,
                        rows_v, rest[:NB], rest[NB:2 * NB], rest[2 * NB])

    return _agg_kernel


def kernel_body_agg(g_hbm, src_hbm, dst_hbm, out_hbm, sidx_v, didx_v, rows_v,
                    gsem, ssem, acc_sh):
    cid = lax.axis_index("c")
    sid = lax.axis_index("s")
    wid = cid * NS + sid

    def buf(k):
        return rows_v.at[pl.ds(k * CHUNK, CHUNK)]

    @pl.loop(0, CHUNK)
    def _(r):
        @pl.loop(0, HID, step=16)
        def _(c0):
            rows_v[r, pl.ds(c0, 16)] = jnp.zeros((16,), jnp.float32)

    @pl.loop(0, RPT // CHUNK)
    def _(k):
        pltpu.sync_copy(buf(0), acc_sh.at[pl.ds(sid * RPT + k * CHUNK, CHUNK)])

    pltpu.sync_copy(src_hbm.at[wid], sidx_v)
    pltpu.sync_copy(dst_hbm.at[wid], didx_v)

    for k in range(NB):
        pltpu.async_copy(g_hbm.at[sidx_v.at[k]], buf(k), gsem[k])

    plsc.subcore_barrier()

    @pl.loop(0, CPT // NB)
    def _(t):
        base = t * NB
        for k in range(NB):
            pltpu.make_async_copy(g_hbm.at[sidx_v.at[0]], buf(k),
                                  gsem[k]).wait()
            pltpu.async_copy(buf(k), acc_sh.at[didx_v.at[base + k]],
                             ssem[k], add=True)
        for k in range(NB):
            @pl.when(t < CPT // NB - 1)
            def _():
                pltpu.make_async_copy(buf(k), acc_sh.at[didx_v.at[0]],
                                      ssem[k]).wait()
                pltpu.async_copy(g_hbm.at[sidx_v.at[base + NB + k]],
                                 buf(k), gsem[k])

    for k in range(NB):
        pltpu.make_async_copy(buf(k), acc_sh.at[didx_v.at[0]], ssem[k]).wait()

    plsc.subcore_barrier()
    pltpu.sync_copy(acc_sh.at[pl.ds(sid * RPT, RPT)],
                    out_hbm.at[cid].at[pl.ds(sid * RPT, RPT)])


def _row_ids(i):
    return i * BLK + lax.broadcasted_iota(jnp.int32, (BLK, 1), 0)


def _tca_body(x_ref, w1_ref, h1_ref):
    i = pl.program_id(0)
    h = lax.dot_general(x_ref[...], w1_ref[...],
                        (((1,), (1,)), ((), ())),
                        preferred_element_type=jnp.float32)
    h1_ref[...] = jnp.where(_row_ids(i) < N, h, 0.0)


def _tcb_body(deg_ref, h1_ref, g1_ref, dinv_ref):
    deg = deg_ref[0, :, :1] + deg_ref[1, :, :1] + 1.0
    dinv = lax.rsqrt(deg)
    dinv_ref[...] = jnp.broadcast_to(dinv, (BLK, 16))
    g1_ref[...] = h1_ref[...] * dinv


def _tc2_body(acc_ref, g1_ref, dinv_ref, b1_ref, w2_ref, g2_ref):
    dinv = dinv_ref[:, :1]
    z1 = jnp.maximum(dinv * (acc_ref[0] + acc_ref[1] + g1_ref[...])
                     + b1_ref[...], 0.0)
    h2 = lax.dot_general(z1, w2_ref[...], (((1,), (1,)), ((), ())),
                         preferred_element_type=jnp.float32)
    g2_ref[...] = h2 * dinv


def _tc3_body(acc_ref, g2_ref, dinv_ref, b2_ref, batch_ref,
              wfc_ref, bfc_ref, out_ref, pool_ref):
    i = pl.program_id(0)
    dinv = dinv_ref[:, :1]
    z2 = jnp.maximum(dinv * (acc_ref[0] + acc_ref[1] + g2_ref[...])
                     + b2_ref[...], 0.0)
    valid = _row_ids(i) < N
    onehot = jnp.where(
        valid,
        (batch_ref[...] == jnp.arange(G, dtype=jnp.int32)[None, :]
         ).astype(jnp.float32),
        0.0)
    ext = jnp.concatenate([z2, jnp.ones((BLK, 1), jnp.float32)], axis=1)
    part = lax.dot_general(onehot, ext, (((0,), (0,)), ((), ())),
                           preferred_element_type=jnp.float32)

    @pl.when(i == 0)
    def _():
        pool_ref[...] = part

    @pl.when(i > 0)
    def _():
        pool_ref[...] += part

    @pl.when(i == NP // BLK - 1)
    def _():
        sums = pool_ref[:, :HID]
        cnt = jnp.maximum(pool_ref[:, HID:HID + 1], 1.0)
        pooled = sums / cnt
        out_ref[...] = lax.dot_general(
            pooled, wfc_ref[...], (((1,), (1,)), ((), ())),
            preferred_element_type=jnp.float32) + bfc_ref[...]


def _row_spec(cols):
    return pl.BlockSpec((BLK, cols), lambda i: (i, 0))


def _pair_spec(cols):
    return pl.BlockSpec((NC, BLK, cols), lambda i: (0, i, 0))


def _full_spec(shape):
    return pl.BlockSpec(shape, lambda i: tuple(0 for _ in shape))


def _tca(x, w1):
    return pl.pallas_call(
        _tca_body,
        grid=(NP // BLK,),
        in_specs=[_row_spec(IN_F), _full_spec((HID, IN_F))],
        out_specs=_row_spec(HID),
        out_shape=jax.ShapeDtypeStruct((NP, HID), jnp.float32),
    )(x, w1)


def _tcb(deg, h1):
    return pl.pallas_call(
        _tcb_body,
        grid=(NP // BLK,),
        in_specs=[_pair_spec(16), _row_spec(HID)],
        out_specs=[_row_spec(HID), _row_spec(16)],
        out_shape=[jax.ShapeDtypeStruct((NP, HID), jnp.float32),
                   jax.ShapeDtypeStruct((NP, 16), jnp.float32)],
    )(deg, h1)


def _tc2(acc, g1, dinv16, b1, w2):
    return pl.pallas_call(
        _tc2_body,
        grid=(NP // BLK,),
        in_specs=[_pair_spec(HID), _row_spec(HID),
                  _row_spec(16), _full_spec((1, HID)), _full_spec((HID, HID))],
        out_specs=_row_spec(HID),
        out_shape=jax.ShapeDtypeStruct((NP, HID), jnp.float32),
    )(acc, g1, dinv16, b1, w2)


def _tc3(acc, g2, dinv16, b2, batch_col, wfc, bfc):
    return pl.pallas_call(
        _tc3_body,
        grid=(NP // BLK,),
        in_specs=[_pair_spec(HID), _row_spec(HID),
                  _row_spec(16), _full_spec((1, HID)), _row_spec(1),
                  _full_spec((N_CLS, HID)), _full_spec((1, N_CLS))],
        out_specs=_full_spec((G, N_CLS)),
        out_shape=jax.ShapeDtypeStruct((G, N_CLS), jnp.float32),
        scratch_shapes=[pltpu.VMEM((G, HID + 1), jnp.float32)],
    )(acc, g2, dinv16, b2, batch_col, wfc, bfc)


def kernel(x, edge_index, batch, W1, b1, W2, b2, Wfc, bfc):
    pad_idx = N + (jnp.arange(E_PAD - E, dtype=jnp.int32) % (NP - N))
    src_p = jnp.concatenate(
        [edge_index[0].astype(jnp.int32), pad_idx]).reshape(NW, CPT, CHUNK)
    dst_p = jnp.concatenate(
        [edge_index[1].astype(jnp.int32), pad_idx]).reshape(NW, CPT, CHUNK)
    batch_col = batch.astype(jnp.int32).reshape(N, 1)

    h1 = _tca(x, W1)
    deg = _make_deg_kernel()(dst_p)
    g1, dinv16 = _tcb(deg, h1)
    acc1 = _make_agg_kernel()(g1, src_p, dst_p)
    g2 = _tc2(acc1, g1, dinv16, b1.reshape(1, HID), W2)
    acc2 = _make_agg_kernel()(g2, src_p, dst_p)
    out = _tc3(acc2, g2, dinv16, b2.reshape(1, HID), batch_col,
               Wfc, bfc.reshape(1, N_CLS))
    return out
